# baseline (speedup 1.0000x reference)
import jax
import jax.numpy as jnp
from jax.experimental import pallas as pl
from jax.experimental.pallas import tpu as pltpu

_BLOCK = 8192


def _copy_block(emb_ref, out_ref):
    out_ref[0, :, :] = emb_ref[:, :]


def kernel(x, emb):
    T = x.shape[1]
    D = emb.shape[1]
    assert T % _BLOCK == 0
    out = pl.pallas_call(
        _copy_block,
        grid=(T // _BLOCK,),
        in_specs=[pl.BlockSpec((_BLOCK, D), lambda i: (i, 0))],
        out_specs=pl.BlockSpec((1, _BLOCK, D), lambda i: (0, i, 0)),
        out_shape=jax.ShapeDtypeStruct((1, T, D), emb.dtype),
        compiler_params=pltpu.CompilerParams(
            dimension_semantics=("parallel",),
        ),
    )(emb[:T])
    return out


# speedup vs baseline: 1.1024x; 1.1024x over previous
import jax
import jax.numpy as jnp
from jax.experimental import pallas as pl
from jax.experimental.pallas import tpu as pltpu

_N_CHUNKS = 16


def _dma_pipe(emb_ref, out_ref, buf, in_sems, out_sems):
    rows = emb_ref.shape[0]
    chunk = rows // _N_CHUNKS

    def in_copy(i):
        return pltpu.make_async_copy(
            emb_ref.at[pl.ds(i * chunk, chunk), :], buf.at[i], in_sems.at[i]
        )

    def out_copy(i):
        return pltpu.make_async_copy(
            buf.at[i], out_ref.at[0, pl.ds(i * chunk, chunk), :], out_sems.at[i]
        )

    for i in range(_N_CHUNKS):
        in_copy(i).start()
    for i in range(_N_CHUNKS):
        in_copy(i).wait()
        out_copy(i).start()
    for i in range(_N_CHUNKS):
        out_copy(i).wait()


def kernel(x, emb):
    T = x.shape[1]
    D = emb.shape[1]
    assert T % _N_CHUNKS == 0
    out = pl.pallas_call(
        _dma_pipe,
        in_specs=[pl.BlockSpec(memory_space=pl.ANY)],
        out_specs=pl.BlockSpec(memory_space=pl.ANY),
        out_shape=jax.ShapeDtypeStruct((1, T, D), emb.dtype),
        scratch_shapes=[
            pltpu.VMEM((_N_CHUNKS, T // _N_CHUNKS, D), emb.dtype),
            pltpu.SemaphoreType.DMA((_N_CHUNKS,)),
            pltpu.SemaphoreType.DMA((_N_CHUNKS,)),
        ],
    )(emb[:T])
    return out


# speedup vs baseline: 1.1195x; 1.0156x over previous
import jax
import jax.numpy as jnp
from jax.experimental import pallas as pl
from jax.experimental.pallas import tpu as pltpu

_N_CHUNKS = 4


def _dma_pipe(emb_ref, out_ref, buf, in_sems, out_sems):
    rows = emb_ref.shape[0]
    chunk = rows // _N_CHUNKS

    def in_copy(i):
        return pltpu.make_async_copy(
            emb_ref.at[pl.ds(i * chunk, chunk), :], buf.at[i], in_sems.at[i]
        )

    def out_copy(i):
        return pltpu.make_async_copy(
            buf.at[i], out_ref.at[0, pl.ds(i * chunk, chunk), :], out_sems.at[i]
        )

    for i in range(_N_CHUNKS):
        in_copy(i).start()
    for i in range(_N_CHUNKS):
        in_copy(i).wait()
        out_copy(i).start()
    for i in range(_N_CHUNKS):
        out_copy(i).wait()


def kernel(x, emb):
    T = x.shape[1]
    D = emb.shape[1]
    assert T % _N_CHUNKS == 0
    out = pl.pallas_call(
        _dma_pipe,
        in_specs=[pl.BlockSpec(memory_space=pl.ANY)],
        out_specs=pl.BlockSpec(memory_space=pl.ANY),
        out_shape=jax.ShapeDtypeStruct((1, T, D), emb.dtype),
        scratch_shapes=[
            pltpu.VMEM((_N_CHUNKS, T // _N_CHUNKS, D), emb.dtype),
            pltpu.SemaphoreType.DMA((_N_CHUNKS,)),
            pltpu.SemaphoreType.DMA((_N_CHUNKS,)),
        ],
    )(emb[:T])
    return out


# speedup vs baseline: 1.1422x; 1.0203x over previous
import jax
import jax.numpy as jnp
from jax.experimental import pallas as pl
from jax.experimental.pallas import tpu as pltpu

_N_CHUNKS = 2


def _dma_pipe(emb_ref, out_ref, buf, in_sems, out_sems):
    rows = emb_ref.shape[0]
    chunk = rows // _N_CHUNKS

    def in_copy(i):
        return pltpu.make_async_copy(
            emb_ref.at[pl.ds(i * chunk, chunk), :], buf.at[i], in_sems.at[i]
        )

    def out_copy(i):
        return pltpu.make_async_copy(
            buf.at[i], out_ref.at[0, pl.ds(i * chunk, chunk), :], out_sems.at[i]
        )

    for i in range(_N_CHUNKS):
        in_copy(i).start()
    for i in range(_N_CHUNKS):
        in_copy(i).wait()
        out_copy(i).start()
    for i in range(_N_CHUNKS):
        out_copy(i).wait()


def kernel(x, emb):
    T = x.shape[1]
    D = emb.shape[1]
    assert T % _N_CHUNKS == 0
    out = pl.pallas_call(
        _dma_pipe,
        in_specs=[pl.BlockSpec(memory_space=pl.ANY)],
        out_specs=pl.BlockSpec(memory_space=pl.ANY),
        out_shape=jax.ShapeDtypeStruct((1, T, D), emb.dtype),
        scratch_shapes=[
            pltpu.VMEM((_N_CHUNKS, T // _N_CHUNKS, D), emb.dtype),
            pltpu.SemaphoreType.DMA((_N_CHUNKS,)),
            pltpu.SemaphoreType.DMA((_N_CHUNKS,)),
        ],
    )(emb[:T])
    return out
